# Initial kernel scaffold; baseline (speedup 1.0000x reference)
import jax
import jax.numpy as jnp
from jax.experimental import pallas as pl


def kernel(x, embed_weight, pe):
    raise NotImplementedError("write your pallas kernel here")


# speedup vs baseline: 2.9220x; 2.9220x over previous
import functools

import jax
import jax.numpy as jnp
from jax import lax
from jax.experimental import pallas as pl
from jax.experimental.pallas import tpu as pltpu
from jax.experimental.pallas import tpu_sc as plsc

_N_TOK = 39
_N_POS = 50
_D = 32
_ROWS = 16384 * _N_POS
_CHUNK = 128
_N_CHUNKS = _ROWS // _CHUNK


def _prep_body(x_ref, e_ref, pe_ref, idx_ref, tbl_ref):
    tbl_ref[...] = e_ref[...][:, None, :] + pe_ref[...]
    pos = lax.broadcasted_iota(jnp.int32, x_ref.shape, 1)
    idx_ref[...] = x_ref[...] * _N_POS + pos


def _sc_gather(tbl, idx2d):
    info = plsc.get_sparse_core_info()
    nw = info.num_cores * info.num_subcores
    chunks_w = _N_CHUNKS // nw
    rows_w = _ROWS // nw
    grp = 10
    n_pairs = chunks_w // (2 * grp)
    buf_rows = grp * _CHUNK

    mesh = plsc.VectorSubcoreMesh(core_axis_name="c", subcore_axis_name="s")

    @functools.partial(
        pl.kernel,
        mesh=mesh,
        out_type=jax.ShapeDtypeStruct((_ROWS, _D), jnp.float32),
        compiler_params=pltpu.CompilerParams(use_tc_tiling_on_sc=False),
        scratch_types=[
            pltpu.VMEM((chunks_w, _CHUNK), jnp.int32),
            pltpu.VMEM((buf_rows, _D), jnp.float32),
            pltpu.VMEM((buf_rows, _D), jnp.float32),
            pltpu.SemaphoreType.DMA,
            pltpu.SemaphoreType.DMA,
            pltpu.SemaphoreType.DMA,
            pltpu.SemaphoreType.DMA,
        ],
    )
    def k(tbl_hbm, idx_hbm, out_hbm, idx_v, buf_a, buf_b, gsem_a, gsem_b,
          ssem_a, ssem_b):
        wid = lax.axis_index("s") * info.num_cores + lax.axis_index("c")
        pltpu.sync_copy(idx_hbm.at[pl.ds(wid * chunks_w, chunks_w)], idx_v)
        row0 = wid * rows_w

        def body(t, _):
            c0 = t * 2 * grp
            d_a = [
                pltpu.async_copy(
                    tbl_hbm.at[idx_v.at[c0 + i]],
                    buf_a.at[pl.ds(i * _CHUNK, _CHUNK)], gsem_a)
                for i in range(grp)
            ]
            d_b = [
                pltpu.async_copy(
                    tbl_hbm.at[idx_v.at[c0 + grp + i]],
                    buf_b.at[pl.ds(i * _CHUNK, _CHUNK)], gsem_b)
                for i in range(grp)
            ]
            for d in d_a:
                d.wait()
            s_a = pltpu.async_copy(
                buf_a, out_hbm.at[pl.ds(row0 + c0 * _CHUNK, buf_rows)], ssem_a)
            for d in d_b:
                d.wait()
            s_b = pltpu.async_copy(
                buf_b,
                out_hbm.at[pl.ds(row0 + c0 * _CHUNK + buf_rows, buf_rows)],
                ssem_b)
            s_a.wait()
            s_b.wait()
            return 0

        lax.fori_loop(0, n_pairs, body, 0)

    return k(tbl, idx2d)


def kernel(x, embed_weight, pe):
    x = x.astype(jnp.int32)
    idx, tbl3 = pl.pallas_call(
        _prep_body,
        out_shape=(
            jax.ShapeDtypeStruct((16384, _N_POS), jnp.int32),
            jax.ShapeDtypeStruct((_N_TOK, _N_POS, _D), jnp.float32),
        ),
    )(x, embed_weight, pe)
    tbl = tbl3.reshape(_N_TOK * _N_POS, _D)
    idx2d = idx.reshape(_N_CHUNKS, _CHUNK)
    out = _sc_gather(tbl, idx2d)
    return out.reshape(16384, _N_POS, _D)
